# baseline (speedup 1.0000x reference)
import functools

import jax
import jax.numpy as jnp
from jax import lax
from jax.experimental import pallas as pl
from jax.experimental.pallas import tpu as pltpu
from jax.experimental.pallas import tpu_sc as plsc

_K = 64
_G = 32
_NC = 2
_NS = 16
_NW = _NC * _NS


def _round_up(x, m):
    return (x + m - 1) // m * m


@functools.lru_cache(maxsize=None)
def _make_sc_features(N, D, E):
    EPT = E // _NW
    CH = EPT // _K
    NG = CH // _G
    assert CH * _K == EPT and EPT * _NW == E and NG * _G == CH
    RPT = N // _NS
    assert RPT % _K == 0
    NB = RPT // _K
    GE = _G * _K
    f32 = jnp.float32
    mesh = plsc.VectorSubcoreMesh(core_axis_name="c", subcore_axis_name="s")
    out_type = [jax.ShapeDtypeStruct((_NC, N, D), f32)] * 3

    def body(src_f, dst_f, src_b, dst_b, src_rb, dst_rb,
             x_user, x_item, zeros_feat,
             agg_f, agg_b, agg_rb,
             acc, sbulk, dbulk, rows0, rows1, dstg0, dstg1,
             sem_g0, sem_g1, sem_s0, sem_s1):
        c = lax.axis_index("c")
        s = lax.axis_index("s")
        wid = c * _NS + s
        r0 = s * RPT
        dummy = zeros_feat.at[pl.ds(0, _K)]

        def stage(dstg, q):
            for g in range(_K // 16):
                dstg[pl.ds(g * 16, 16)] = dbulk[pl.ds(q * _K + g * 16, 16)]

        def gather_start(table, rows, sem, q):
            pltpu.async_copy(
                table.at[sbulk.at[pl.ds(pl.multiple_of(q * _K, 8), _K)]],
                rows, sem)

        for srcs, dsts, table, agg_out in (
                (src_f, dst_f, x_user, agg_f),
                (src_b, dst_b, x_user, agg_b),
                (src_rb, dst_rb, x_item, agg_rb)):
            pltpu.sync_copy(zeros_feat.at[pl.ds(0, _K)], rows0)
            for z in range(NB):
                pltpu.sync_copy(rows0, acc.at[pl.ds(r0 + z * _K, _K)])
            plsc.subcore_barrier()
            base = wid * EPT

            for m in range(NG):
                goff = pl.multiple_of(base + m * GE, 8)
                pltpu.sync_copy(srcs.at[pl.ds(goff, GE)], sbulk)
                pltpu.sync_copy(dsts.at[pl.ds(goff, GE)], dbulk)
                gather_start(table, rows0, sem_g0, 0)

                def inner(j2, carry):
                    q0 = j2 * 2

                    pltpu.make_async_copy(dummy, rows0, sem_g0).wait()
                    stage(dstg0, q0)

                    @pl.when(q0 > 0)
                    def _():
                        pltpu.make_async_copy(dummy, rows1, sem_s1).wait()

                    gather_start(table, rows1, sem_g1, q0 + 1)
                    pltpu.async_copy(rows0, acc.at[dstg0], sem_s0, add=True)

                    pltpu.make_async_copy(dummy, rows1, sem_g1).wait()
                    stage(dstg1, q0 + 1)
                    pltpu.make_async_copy(dummy, rows0, sem_s0).wait()

                    @pl.when(q0 + 2 < _G)
                    def _():
                        gather_start(table, rows0, sem_g0, q0 + 2)

                    pltpu.async_copy(rows1, acc.at[dstg1], sem_s1, add=True)
                    return carry

                lax.fori_loop(0, _G // 2, inner, 0)
                pltpu.make_async_copy(dummy, rows1, sem_s1).wait()

            plsc.subcore_barrier()
            for z in range(NB):
                pltpu.sync_copy(acc.at[pl.ds(r0 + z * _K, _K)], rows0)
                pltpu.sync_copy(rows0, agg_out.at[c, pl.ds(r0 + z * _K, _K)])

    return pl.kernel(
        body,
        out_type=out_type,
        mesh=mesh,
        compiler_params=pltpu.CompilerParams(needs_layout_passes=False),
        scratch_types=[
            pltpu.VMEM_SHARED((N, D), f32),
            pltpu.VMEM((_G * _K,), jnp.int32),
            pltpu.VMEM((_G * _K,), jnp.int32),
            pltpu.VMEM((_K, D), f32),
            pltpu.VMEM((_K, D), f32),
            pltpu.VMEM((_K,), jnp.int32),
            pltpu.VMEM((_K,), jnp.int32),
            pltpu.SemaphoreType.DMA,
            pltpu.SemaphoreType.DMA,
            pltpu.SemaphoreType.DMA,
            pltpu.SemaphoreType.DMA,
        ],
    )


@functools.lru_cache(maxsize=None)
def _make_sc_degrees(N, E):
    EPT = E // _NW
    NGRP = EPT // 16
    assert NGRP * 16 == EPT
    f32 = jnp.float32
    mesh = plsc.VectorSubcoreMesh(core_axis_name="c", subcore_axis_name="s")
    out_type = [jax.ShapeDtypeStruct((_NW * N,), f32)] * 3

    def body(dst_f, dst_b, dst_rb, zeros_deg,
             deg_f, deg_b, deg_rb,
             deg, dbulk):
        c = lax.axis_index("c")
        s = lax.axis_index("s")
        wid = c * _NS + s
        base = wid * EPT
        ones16 = jnp.ones((16,), f32)
        NGB = 2048
        for dsts, deg_out in ((dst_f, deg_f), (dst_b, deg_b),
                              (dst_rb, deg_rb)):
            pltpu.sync_copy(zeros_deg, deg)
            for m in range(EPT // NGB):
                pltpu.sync_copy(
                    dsts.at[pl.ds(pl.multiple_of(base + m * NGB, 8), NGB)],
                    dbulk)

                def grp(g, carry):
                    iv = dbulk[pl.ds(g * 16, 16)]
                    plsc.addupdate_scatter(deg, [iv], ones16)
                    return carry

                lax.fori_loop(0, NGB // 16, grp, 0)
            pltpu.sync_copy(deg, deg_out.at[pl.ds(wid * N, N)])

    return pl.kernel(
        body,
        out_type=out_type,
        mesh=mesh,
        compiler_params=pltpu.CompilerParams(needs_layout_passes=False),
        scratch_types=[
            pltpu.VMEM((N,), f32),
            pltpu.VMEM((2048,), jnp.int32),
        ],
    )


@functools.lru_cache(maxsize=None)
def _make_epilogue(N, Np, D, nrel):
    f32 = jnp.float32
    R = 128

    def body(*args):
        o_ref = args[-1]
        ones = jnp.ones((_NW, 1), f32)
        for r in range(nrel):
            a_ref, d_ref, w_ref = args[3 * r], args[3 * r + 1], args[3 * r + 2]
            p = a_ref[0] + a_ref[1]
            d = d_ref[:, 0, 0, :]
            dcol = lax.dot_general(d, ones, (((0,), (0,)), ((), ())),
                                   preferred_element_type=f32)
            dcol = jnp.maximum(dcol, 1.0)
            o_ref[:, r, :] = jnp.dot(p / dcol, w_ref[...],
                                     preferred_element_type=f32)

    in_specs = []
    for _ in range(nrel):
        in_specs += [
            pl.BlockSpec((_NC, R, D), lambda i: (0, i, 0)),
            pl.BlockSpec((_NW, 1, 1, 128), lambda i: (0, i, 0, 0)),
            pl.BlockSpec((D, D), lambda i: (0, 0)),
        ]
    grid = (pl.cdiv(N, R),)
    return pl.pallas_call(
        body,
        grid=grid,
        in_specs=in_specs,
        out_specs=pl.BlockSpec((R, nrel, D), lambda i: (i, 0, 0)),
        out_shape=jax.ShapeDtypeStruct((N, nrel, D), f32),
    )


def kernel(x_user, x_item, edge_index_follows, edge_index_buys,
           edge_index_rev_buys, W_follows, W_buys, W_rev_buys):
    N, D = x_user.shape
    E = edge_index_follows.shape[1]
    i32 = jnp.int32
    f32 = jnp.float32
    Np = _round_up(N, _K * _NS)
    Ep = _round_up(E, _K * _G * _NW)
    npad = Ep - E
    pad_src = jnp.zeros((npad,), i32)
    pad_dst = N + (jnp.arange(npad, dtype=i32) % (Np - N))

    def prep(ei):
        return (jnp.concatenate([ei[0].astype(i32), pad_src]),
                jnp.concatenate([ei[1].astype(i32), pad_dst]))

    src_f, dst_f = prep(edge_index_follows)
    src_b, dst_b = prep(edge_index_buys)
    src_rb, dst_rb = prep(edge_index_rev_buys)
    zeros_feat = jnp.zeros((Np, D), f32)
    zeros_deg = jnp.zeros((Np,), f32)

    agg_f, agg_b, agg_rb = _make_sc_features(Np, D, Ep)(
        src_f, dst_f, src_b, dst_b, src_rb, dst_rb,
        x_user.astype(f32), x_item.astype(f32), zeros_feat)
    deg_f, deg_b, deg_rb = _make_sc_degrees(Np, Ep)(
        dst_f, dst_b, dst_rb, zeros_deg)
    deg_f, deg_b, deg_rb = (d.reshape(_NW, Np // 128, 1, 128)
                            for d in (deg_f, deg_b, deg_rb))

    out_user = _make_epilogue(N, Np, D, 2)(
        agg_f, deg_f, W_follows, agg_rb, deg_rb, W_rev_buys)
    out_item = _make_epilogue(N, Np, D, 1)(agg_b, deg_b, W_buys)
    return out_user, out_item


# speedup vs baseline: 1.0013x; 1.0013x over previous
import functools

import jax
import jax.numpy as jnp
from jax import lax
from jax.experimental import pallas as pl
from jax.experimental.pallas import tpu as pltpu
from jax.experimental.pallas import tpu_sc as plsc

_K = 64
_G = 32
_NC = 2
_NS = 16
_NW = _NC * _NS


def _round_up(x, m):
    return (x + m - 1) // m * m


@functools.lru_cache(maxsize=None)
def _make_sc_features(N, D, E):
    EPT = E // _NW
    CH = EPT // _K
    NG = CH // _G
    assert CH * _K == EPT and EPT * _NW == E and NG * _G == CH
    RPT = N // _NS
    assert RPT % _K == 0
    NB = RPT // _K
    GE = _G * _K
    f32 = jnp.float32
    mesh = plsc.VectorSubcoreMesh(core_axis_name="c", subcore_axis_name="s")
    out_type = [jax.ShapeDtypeStruct((_NC, N, D), f32)] * 3

    def body(src_f, dst_f, src_b, dst_b, src_rb, dst_rb,
             x_user, x_item, zeros_feat,
             agg_f, agg_b, agg_rb,
             acc, sbulk, dbulk, rows0, rows1, dstg0, dstg1,
             sem_g0, sem_g1):
        c = lax.axis_index("c")
        s = lax.axis_index("s")
        wid = c * _NS + s
        r0 = s * RPT
        dummy = zeros_feat.at[pl.ds(0, _K)]

        def stage(dstg, q):
            for g in range(_K // 16):
                dstg[pl.ds(g * 16, 16)] = dbulk[pl.ds(q * _K + g * 16, 16)]

        def gather_start(table, rows, sem, q):
            pltpu.async_copy(
                table.at[sbulk.at[pl.ds(pl.multiple_of(q * _K, 8), _K)]],
                rows, sem)

        for srcs, dsts, table, agg_out in (
                (src_f, dst_f, x_user, agg_f),
                (src_b, dst_b, x_user, agg_b),
                (src_rb, dst_rb, x_item, agg_rb)):
            pltpu.sync_copy(zeros_feat.at[pl.ds(0, _K)], rows0)
            for z in range(NB):
                pltpu.sync_copy(rows0, acc.at[pl.ds(r0 + z * _K, _K)])
            plsc.subcore_barrier()
            base = wid * EPT

            for m in range(NG):
                goff = pl.multiple_of(base + m * GE, 8)
                pltpu.sync_copy(srcs.at[pl.ds(goff, GE)], sbulk)
                pltpu.sync_copy(dsts.at[pl.ds(goff, GE)], dbulk)
                gather_start(table, rows0, sem_g0, 0)

                def inner(j2, carry):
                    q0 = j2 * 2

                    pltpu.make_async_copy(dummy, rows0, sem_g0).wait()
                    stage(dstg0, q0)
                    gather_start(table, rows1, sem_g1, q0 + 1)
                    pltpu.sync_copy(rows0, acc.at[dstg0], add=True)

                    pltpu.make_async_copy(dummy, rows1, sem_g1).wait()
                    stage(dstg1, q0 + 1)

                    @pl.when(q0 + 2 < _G)
                    def _():
                        gather_start(table, rows0, sem_g0, q0 + 2)

                    pltpu.sync_copy(rows1, acc.at[dstg1], add=True)
                    return carry

                lax.fori_loop(0, _G // 2, inner, 0)

            plsc.subcore_barrier()
            for z in range(NB):
                pltpu.sync_copy(acc.at[pl.ds(r0 + z * _K, _K)], rows0)
                pltpu.sync_copy(rows0, agg_out.at[c, pl.ds(r0 + z * _K, _K)])

    return pl.kernel(
        body,
        out_type=out_type,
        mesh=mesh,
        compiler_params=pltpu.CompilerParams(needs_layout_passes=False),
        scratch_types=[
            pltpu.VMEM_SHARED((N, D), f32),
            pltpu.VMEM((_G * _K,), jnp.int32),
            pltpu.VMEM((_G * _K,), jnp.int32),
            pltpu.VMEM((_K, D), f32),
            pltpu.VMEM((_K, D), f32),
            pltpu.VMEM((_K,), jnp.int32),
            pltpu.VMEM((_K,), jnp.int32),
            pltpu.SemaphoreType.DMA,
            pltpu.SemaphoreType.DMA,
        ],
    )


@functools.lru_cache(maxsize=None)
def _make_sc_degrees(N, E):
    EPT = E // _NW
    NGRP = EPT // 16
    assert NGRP * 16 == EPT
    f32 = jnp.float32
    mesh = plsc.VectorSubcoreMesh(core_axis_name="c", subcore_axis_name="s")
    out_type = [jax.ShapeDtypeStruct((_NW * N,), f32)] * 3

    def body(dst_f, dst_b, dst_rb, zeros_deg,
             deg_f, deg_b, deg_rb,
             deg, dbulk):
        c = lax.axis_index("c")
        s = lax.axis_index("s")
        wid = c * _NS + s
        base = wid * EPT
        ones16 = jnp.ones((16,), f32)
        NGB = 2048
        for dsts, deg_out in ((dst_f, deg_f), (dst_b, deg_b),
                              (dst_rb, deg_rb)):
            pltpu.sync_copy(zeros_deg, deg)
            for m in range(EPT // NGB):
                pltpu.sync_copy(
                    dsts.at[pl.ds(pl.multiple_of(base + m * NGB, 8), NGB)],
                    dbulk)

                def grp(g, carry):
                    iv = dbulk[pl.ds(g * 16, 16)]
                    plsc.addupdate_scatter(deg, [iv], ones16)
                    return carry

                lax.fori_loop(0, NGB // 16, grp, 0)
            pltpu.sync_copy(deg, deg_out.at[pl.ds(wid * N, N)])

    return pl.kernel(
        body,
        out_type=out_type,
        mesh=mesh,
        compiler_params=pltpu.CompilerParams(needs_layout_passes=False),
        scratch_types=[
            pltpu.VMEM((N,), f32),
            pltpu.VMEM((2048,), jnp.int32),
        ],
    )


@functools.lru_cache(maxsize=None)
def _make_epilogue(N, Np, D, nrel):
    f32 = jnp.float32
    R = 128

    def body(*args):
        o_ref = args[-1]
        ones = jnp.ones((_NW, 1), f32)
        for r in range(nrel):
            a_ref, d_ref, w_ref = args[3 * r], args[3 * r + 1], args[3 * r + 2]
            p = a_ref[0] + a_ref[1]
            d = d_ref[:, 0, 0, :]
            dcol = lax.dot_general(d, ones, (((0,), (0,)), ((), ())),
                                   preferred_element_type=f32)
            dcol = jnp.maximum(dcol, 1.0)
            o_ref[:, r, :] = jnp.dot(p / dcol, w_ref[...],
                                     preferred_element_type=f32)

    in_specs = []
    for _ in range(nrel):
        in_specs += [
            pl.BlockSpec((_NC, R, D), lambda i: (0, i, 0)),
            pl.BlockSpec((_NW, 1, 1, 128), lambda i: (0, i, 0, 0)),
            pl.BlockSpec((D, D), lambda i: (0, 0)),
        ]
    grid = (pl.cdiv(N, R),)
    return pl.pallas_call(
        body,
        grid=grid,
        in_specs=in_specs,
        out_specs=pl.BlockSpec((R, nrel, D), lambda i: (i, 0, 0)),
        out_shape=jax.ShapeDtypeStruct((N, nrel, D), f32),
    )


def kernel(x_user, x_item, edge_index_follows, edge_index_buys,
           edge_index_rev_buys, W_follows, W_buys, W_rev_buys):
    N, D = x_user.shape
    E = edge_index_follows.shape[1]
    i32 = jnp.int32
    f32 = jnp.float32
    Np = _round_up(N, _K * _NS)
    Ep = _round_up(E, _K * _G * _NW)
    npad = Ep - E
    pad_src = jnp.zeros((npad,), i32)
    pad_dst = N + (jnp.arange(npad, dtype=i32) % (Np - N))

    def prep(ei):
        return (jnp.concatenate([ei[0].astype(i32), pad_src]),
                jnp.concatenate([ei[1].astype(i32), pad_dst]))

    src_f, dst_f = prep(edge_index_follows)
    src_b, dst_b = prep(edge_index_buys)
    src_rb, dst_rb = prep(edge_index_rev_buys)
    zeros_feat = jnp.zeros((Np, D), f32)
    zeros_deg = jnp.zeros((Np,), f32)

    agg_f, agg_b, agg_rb = _make_sc_features(Np, D, Ep)(
        src_f, dst_f, src_b, dst_b, src_rb, dst_rb,
        x_user.astype(f32), x_item.astype(f32), zeros_feat)
    deg_f, deg_b, deg_rb = _make_sc_degrees(Np, Ep)(
        dst_f, dst_b, dst_rb, zeros_deg)
    deg_f, deg_b, deg_rb = (d.reshape(_NW, Np // 128, 1, 128)
                            for d in (deg_f, deg_b, deg_rb))

    out_user = _make_epilogue(N, Np, D, 2)(
        agg_f, deg_f, W_follows, agg_rb, deg_rb, W_rev_buys)
    out_item = _make_epilogue(N, Np, D, 1)(agg_b, deg_b, W_buys)
    return out_user, out_item


# speedup vs baseline: 1.2563x; 1.2547x over previous
import functools

import jax
import jax.numpy as jnp
from jax import lax
from jax.experimental import pallas as pl
from jax.experimental.pallas import tpu as pltpu
from jax.experimental.pallas import tpu_sc as plsc

_K = 128
_NC = 2
_NS = 16
_NW = _NC * _NS


def _round_up(x, m):
    return (x + m - 1) // m * m


@functools.lru_cache(maxsize=None)
def _make_sc_features(N, D, E):
    EPT = E // _NW
    CH = EPT // _K
    assert CH * _K == EPT and EPT * _NW == E
    RPT = N // _NS
    assert RPT % _K == 0
    NB = RPT // _K
    f32 = jnp.float32
    mesh = plsc.VectorSubcoreMesh(core_axis_name="c", subcore_axis_name="s")
    out_type = [jax.ShapeDtypeStruct((_NC, N, D), f32)] * 3

    def body(src_f, dst_f, src_b, dst_b, src_rb, dst_rb,
             x_user, x_item, zeros_feat,
             agg_f, agg_b, agg_rb,
             acc, idx, rows, sem):
        c = lax.axis_index("c")
        s = lax.axis_index("s")
        wid = c * _NS + s
        r0 = s * RPT
        for srcs, dsts, table, agg_out in (
                (src_f, dst_f, x_user, agg_f),
                (src_b, dst_b, x_user, agg_b),
                (src_rb, dst_rb, x_item, agg_rb)):
            pltpu.sync_copy(zeros_feat.at[pl.ds(0, _K)], rows)
            for z in range(NB):
                pltpu.sync_copy(rows, acc.at[pl.ds(r0 + z * _K, _K)])
            plsc.subcore_barrier()
            base = wid * EPT

            def chunk(j, carry):
                off = pl.multiple_of(base + j * _K, 8)
                pltpu.sync_copy(srcs.at[pl.ds(off, _K)], idx.at[0])
                pltpu.sync_copy(dsts.at[pl.ds(off, _K)], idx.at[1])
                pltpu.async_copy(table.at[idx.at[0]], rows, sem).wait()
                pltpu.sync_copy(rows, acc.at[idx.at[1]], add=True)
                return carry

            lax.fori_loop(0, CH, chunk, 0)
            plsc.subcore_barrier()
            for z in range(NB):
                pltpu.sync_copy(acc.at[pl.ds(r0 + z * _K, _K)], rows)
                pltpu.sync_copy(rows, agg_out.at[c, pl.ds(r0 + z * _K, _K)])

    return pl.kernel(
        body,
        out_type=out_type,
        mesh=mesh,
        compiler_params=pltpu.CompilerParams(needs_layout_passes=False),
        scratch_types=[
            pltpu.VMEM_SHARED((N, D), f32),
            pltpu.VMEM((2, _K), jnp.int32),
            pltpu.VMEM((_K, D), f32),
            pltpu.SemaphoreType.DMA,
        ],
    )


@functools.lru_cache(maxsize=None)
def _make_sc_degrees(N, E):
    EPT = E // _NW
    NGRP = EPT // 16
    assert NGRP * 16 == EPT
    f32 = jnp.float32
    mesh = plsc.VectorSubcoreMesh(core_axis_name="c", subcore_axis_name="s")
    out_type = [jax.ShapeDtypeStruct((_NW * N,), f32)] * 3

    def body(dst_f, dst_b, dst_rb, zeros_deg,
             deg_f, deg_b, deg_rb,
             deg, dbulk):
        c = lax.axis_index("c")
        s = lax.axis_index("s")
        wid = c * _NS + s
        base = wid * EPT
        ones16 = jnp.ones((16,), f32)
        NGB = 2048
        for dsts, deg_out in ((dst_f, deg_f), (dst_b, deg_b),
                              (dst_rb, deg_rb)):
            pltpu.sync_copy(zeros_deg, deg)
            for m in range(EPT // NGB):
                pltpu.sync_copy(
                    dsts.at[pl.ds(pl.multiple_of(base + m * NGB, 8), NGB)],
                    dbulk)

                def grp(g, carry):
                    iv = dbulk[pl.ds(g * 16, 16)]
                    plsc.addupdate_scatter(deg, [iv], ones16)
                    return carry

                lax.fori_loop(0, NGB // 16, grp, 0)
            pltpu.sync_copy(deg, deg_out.at[pl.ds(wid * N, N)])

    return pl.kernel(
        body,
        out_type=out_type,
        mesh=mesh,
        compiler_params=pltpu.CompilerParams(needs_layout_passes=False),
        scratch_types=[
            pltpu.VMEM((N,), f32),
            pltpu.VMEM((2048,), jnp.int32),
        ],
    )


@functools.lru_cache(maxsize=None)
def _make_epilogue(N, Np, D, nrel):
    f32 = jnp.float32
    R = 128

    def body(*args):
        o_ref = args[-1]
        ones = jnp.ones((_NW, 1), f32)
        for r in range(nrel):
            a_ref, d_ref, w_ref = args[3 * r], args[3 * r + 1], args[3 * r + 2]
            p = a_ref[0] + a_ref[1]
            d = d_ref[:, 0, 0, :]
            dcol = lax.dot_general(d, ones, (((0,), (0,)), ((), ())),
                                   preferred_element_type=f32)
            dcol = jnp.maximum(dcol, 1.0)
            o_ref[:, r, :] = jnp.dot(p / dcol, w_ref[...],
                                     preferred_element_type=f32)

    in_specs = []
    for _ in range(nrel):
        in_specs += [
            pl.BlockSpec((_NC, R, D), lambda i: (0, i, 0)),
            pl.BlockSpec((_NW, 1, 1, 128), lambda i: (0, i, 0, 0)),
            pl.BlockSpec((D, D), lambda i: (0, 0)),
        ]
    grid = (pl.cdiv(N, R),)
    return pl.pallas_call(
        body,
        grid=grid,
        in_specs=in_specs,
        out_specs=pl.BlockSpec((R, nrel, D), lambda i: (i, 0, 0)),
        out_shape=jax.ShapeDtypeStruct((N, nrel, D), f32),
    )


def kernel(x_user, x_item, edge_index_follows, edge_index_buys,
           edge_index_rev_buys, W_follows, W_buys, W_rev_buys):
    N, D = x_user.shape
    E = edge_index_follows.shape[1]
    i32 = jnp.int32
    f32 = jnp.float32
    Np = _round_up(N, _K * _NS)
    Ep = _round_up(E, _K * _NW)
    npad = Ep - E
    pad_src = jnp.zeros((npad,), i32)
    pad_dst = N + (jnp.arange(npad, dtype=i32) % (Np - N))

    def prep(ei):
        return (jnp.concatenate([ei[0].astype(i32), pad_src]),
                jnp.concatenate([ei[1].astype(i32), pad_dst]))

    src_f, dst_f = prep(edge_index_follows)
    src_b, dst_b = prep(edge_index_buys)
    src_rb, dst_rb = prep(edge_index_rev_buys)
    zeros_feat = jnp.zeros((Np, D), f32)
    zeros_deg = jnp.zeros((Np,), f32)

    agg_f, agg_b, agg_rb = _make_sc_features(Np, D, Ep)(
        src_f, dst_f, src_b, dst_b, src_rb, dst_rb,
        x_user.astype(f32), x_item.astype(f32), zeros_feat)
    deg_f, deg_b, deg_rb = _make_sc_degrees(Np, Ep)(
        dst_f, dst_b, dst_rb, zeros_deg)
    deg_f, deg_b, deg_rb = (d.reshape(_NW, Np // 128, 1, 128)
                            for d in (deg_f, deg_b, deg_rb))

    out_user = _make_epilogue(N, Np, D, 2)(
        agg_f, deg_f, W_follows, agg_rb, deg_rb, W_rev_buys)
    out_item = _make_epilogue(N, Np, D, 1)(agg_b, deg_b, W_buys)
    return out_user, out_item


# speedup vs baseline: 1.5265x; 1.2150x over previous
import functools

import jax
import jax.numpy as jnp
from jax import lax
from jax.experimental import pallas as pl
from jax.experimental.pallas import tpu as pltpu
from jax.experimental.pallas import tpu_sc as plsc

_K = 80
_NC = 2
_NS = 16
_NW = _NC * _NS


def _round_up(x, m):
    return (x + m - 1) // m * m


@functools.lru_cache(maxsize=None)
def _make_sc_features(N, D, E):
    EPT = E // _NW
    CH = EPT // _K
    assert CH * _K == EPT and EPT * _NW == E
    RPT = N // _NS
    assert RPT % _K == 0
    NB = RPT // _K
    f32 = jnp.float32
    mesh = plsc.VectorSubcoreMesh(core_axis_name="c", subcore_axis_name="s")
    out_type = [jax.ShapeDtypeStruct((_NC, N, D), f32)] * 3

    def body(src_f, dst_f, src_b, dst_b, src_rb, dst_rb,
             x_user, x_item, zeros_feat,
             agg_f, agg_b, agg_rb,
             acc, idx, rows, sem):
        c = lax.axis_index("c")
        s = lax.axis_index("s")
        wid = c * _NS + s
        r0 = s * RPT
        for srcs, dsts, table, agg_out in (
                (src_f, dst_f, x_user, agg_f),
                (src_b, dst_b, x_user, agg_b),
                (src_rb, dst_rb, x_item, agg_rb)):
            pltpu.sync_copy(zeros_feat.at[pl.ds(0, _K)], rows)
            for z in range(NB):
                pltpu.sync_copy(rows, acc.at[pl.ds(r0 + z * _K, _K)])
            plsc.subcore_barrier()
            base = wid * EPT

            def chunk(j, carry):
                off = pl.multiple_of(base + j * _K, 8)
                pltpu.sync_copy(srcs.at[pl.ds(off, _K)], idx.at[0])
                pltpu.sync_copy(dsts.at[pl.ds(off, _K)], idx.at[1])
                pltpu.async_copy(table.at[idx.at[0]], rows, sem).wait()
                pltpu.sync_copy(rows, acc.at[idx.at[1]], add=True)
                return carry

            lax.fori_loop(0, CH, chunk, 0)
            plsc.subcore_barrier()
            for z in range(NB):
                pltpu.sync_copy(acc.at[pl.ds(r0 + z * _K, _K)], rows)
                pltpu.sync_copy(rows, agg_out.at[c, pl.ds(r0 + z * _K, _K)])

    return pl.kernel(
        body,
        out_type=out_type,
        mesh=mesh,
        compiler_params=pltpu.CompilerParams(needs_layout_passes=False),
        scratch_types=[
            pltpu.VMEM_SHARED((N, D), f32),
            pltpu.VMEM((2, _K), jnp.int32),
            pltpu.VMEM((_K, D), f32),
            pltpu.SemaphoreType.DMA,
        ],
    )


@functools.lru_cache(maxsize=None)
def _make_sc_degrees(N, E):
    EPT = E // _NW
    NGRP = EPT // 16
    assert NGRP * 16 == EPT
    f32 = jnp.float32
    mesh = plsc.VectorSubcoreMesh(core_axis_name="c", subcore_axis_name="s")
    out_type = [jax.ShapeDtypeStruct((_NW * N,), f32)] * 3

    def body(dst_f, dst_b, dst_rb, zeros_deg,
             deg_f, deg_b, deg_rb,
             deg, dbulk):
        c = lax.axis_index("c")
        s = lax.axis_index("s")
        wid = c * _NS + s
        base = wid * EPT
        ones16 = jnp.ones((16,), f32)
        NGB = 2000
        assert EPT % NGB == 0 and NGB % 16 == 0
        for dsts, deg_out in ((dst_f, deg_f), (dst_b, deg_b),
                              (dst_rb, deg_rb)):
            pltpu.sync_copy(zeros_deg, deg)
            for m in range(EPT // NGB):
                pltpu.sync_copy(
                    dsts.at[pl.ds(pl.multiple_of(base + m * NGB, 8), NGB)],
                    dbulk)

                def grp(g, carry):
                    iv = dbulk[pl.ds(g * 16, 16)]
                    plsc.addupdate_scatter(deg, [iv], ones16)
                    return carry

                lax.fori_loop(0, NGB // 16, grp, 0)
            pltpu.sync_copy(deg, deg_out.at[pl.ds(wid * N, N)])

    return pl.kernel(
        body,
        out_type=out_type,
        mesh=mesh,
        compiler_params=pltpu.CompilerParams(needs_layout_passes=False),
        scratch_types=[
            pltpu.VMEM((N,), f32),
            pltpu.VMEM((2000,), jnp.int32),
        ],
    )


@functools.lru_cache(maxsize=None)
def _make_epilogue(N, Np, D, nrel):
    f32 = jnp.float32
    R = 128

    def body(*args):
        o_ref = args[-1]
        ones = jnp.ones((_NW, 1), f32)
        for r in range(nrel):
            a_ref, d_ref, w_ref = args[3 * r], args[3 * r + 1], args[3 * r + 2]
            p = a_ref[0] + a_ref[1]
            d = d_ref[:, 0, 0, :]
            dcol = lax.dot_general(d, ones, (((0,), (0,)), ((), ())),
                                   preferred_element_type=f32)
            dcol = jnp.maximum(dcol, 1.0)
            o_ref[:, r, :] = jnp.dot(p / dcol, w_ref[...],
                                     preferred_element_type=f32)

    in_specs = []
    for _ in range(nrel):
        in_specs += [
            pl.BlockSpec((_NC, R, D), lambda i: (0, i, 0)),
            pl.BlockSpec((_NW, 1, 1, 128), lambda i: (0, i, 0, 0)),
            pl.BlockSpec((D, D), lambda i: (0, 0)),
        ]
    grid = (pl.cdiv(N, R),)
    return pl.pallas_call(
        body,
        grid=grid,
        in_specs=in_specs,
        out_specs=pl.BlockSpec((R, nrel, D), lambda i: (i, 0, 0)),
        out_shape=jax.ShapeDtypeStruct((N, nrel, D), f32),
    )


def kernel(x_user, x_item, edge_index_follows, edge_index_buys,
           edge_index_rev_buys, W_follows, W_buys, W_rev_buys):
    N, D = x_user.shape
    E = edge_index_follows.shape[1]
    i32 = jnp.int32
    f32 = jnp.float32
    Np = _round_up(N, _K * _NS)
    Ep = _round_up(E, _K * _NW)
    npad = Ep - E
    pad_src = jnp.zeros((npad,), i32)
    pad_dst = N + (jnp.arange(npad, dtype=i32) % (Np - N))

    def prep(ei):
        return (jnp.concatenate([ei[0].astype(i32), pad_src]),
                jnp.concatenate([ei[1].astype(i32), pad_dst]))

    src_f, dst_f = prep(edge_index_follows)
    src_b, dst_b = prep(edge_index_buys)
    src_rb, dst_rb = prep(edge_index_rev_buys)
    zeros_feat = jnp.zeros((Np, D), f32)
    zeros_deg = jnp.zeros((Np,), f32)

    agg_f, agg_b, agg_rb = _make_sc_features(Np, D, Ep)(
        src_f, dst_f, src_b, dst_b, src_rb, dst_rb,
        x_user.astype(f32), x_item.astype(f32), zeros_feat)
    deg_f, deg_b, deg_rb = _make_sc_degrees(Np, Ep)(
        dst_f, dst_b, dst_rb, zeros_deg)
    deg_f, deg_b, deg_rb = (d.reshape(_NW, Np // 128, 1, 128)
                            for d in (deg_f, deg_b, deg_rb))

    out_user = _make_epilogue(N, Np, D, 2)(
        agg_f, deg_f, W_follows, agg_rb, deg_rb, W_rev_buys)
    out_item = _make_epilogue(N, Np, D, 1)(agg_b, deg_b, W_buys)
    return out_user, out_item


# speedup vs baseline: 1.7117x; 1.1213x over previous
import functools

import jax
import jax.numpy as jnp
from jax import lax
from jax.experimental import pallas as pl
from jax.experimental.pallas import tpu as pltpu
from jax.experimental.pallas import tpu_sc as plsc

_K = 80
_NC = 2
_NS = 16
_NW = _NC * _NS


def _round_up(x, m):
    return (x + m - 1) // m * m


@functools.lru_cache(maxsize=None)
def _make_sc_features(N, D, E):
    EPT = E // _NW
    CH = EPT // _K
    assert CH * _K == EPT and EPT * _NW == E
    RPT = N // _NS
    assert RPT % _K == 0
    NB = RPT // _K
    f32 = jnp.float32
    mesh = plsc.VectorSubcoreMesh(core_axis_name="c", subcore_axis_name="s")
    out_type = [jax.ShapeDtypeStruct((_NC, N, D), f32)] * 3

    assert CH % 2 == 0

    def body(src_f, dst_f, src_b, dst_b, src_rb, dst_rb,
             x_user, x_item, zeros_feat,
             agg_f, agg_b, agg_rb,
             acc, idx0, idx1, rows0, rows1, sem0, sem1):
        c = lax.axis_index("c")
        s = lax.axis_index("s")
        wid = c * _NS + s
        r0 = s * RPT
        dummy = zeros_feat.at[pl.ds(0, _K)]

        for srcs, dsts, table, agg_out in (
                (src_f, dst_f, x_user, agg_f),
                (src_b, dst_b, x_user, agg_b),
                (src_rb, dst_rb, x_item, agg_rb)):
            pltpu.sync_copy(zeros_feat.at[pl.ds(0, _K)], rows0)
            for z in range(NB):
                pltpu.sync_copy(rows0, acc.at[pl.ds(r0 + z * _K, _K)])
            plsc.subcore_barrier()
            base = wid * EPT

            def cp_idx_and_gather(idx, rows, sem, q):
                off = pl.multiple_of(base + q * _K, 8)
                pltpu.sync_copy(srcs.at[pl.ds(off, _K)], idx.at[0])
                pltpu.sync_copy(dsts.at[pl.ds(off, _K)], idx.at[1])
                pltpu.async_copy(table.at[idx.at[0]], rows, sem)

            cp_idx_and_gather(idx0, rows0, sem0, 0)

            def inner(j2, carry):
                q0 = j2 * 2
                cp_idx_and_gather(idx1, rows1, sem1, q0 + 1)
                pltpu.make_async_copy(dummy, rows0, sem0).wait()
                pltpu.sync_copy(rows0, acc.at[idx0.at[1]], add=True)

                @pl.when(q0 + 2 < CH)
                def _():
                    cp_idx_and_gather(idx0, rows0, sem0, q0 + 2)

                pltpu.make_async_copy(dummy, rows1, sem1).wait()
                pltpu.sync_copy(rows1, acc.at[idx1.at[1]], add=True)
                return carry

            lax.fori_loop(0, CH // 2, inner, 0)
            plsc.subcore_barrier()
            for z in range(NB):
                pltpu.sync_copy(acc.at[pl.ds(r0 + z * _K, _K)], rows0)
                pltpu.sync_copy(rows0, agg_out.at[c, pl.ds(r0 + z * _K, _K)])

    return pl.kernel(
        body,
        out_type=out_type,
        mesh=mesh,
        compiler_params=pltpu.CompilerParams(needs_layout_passes=False),
        scratch_types=[
            pltpu.VMEM_SHARED((N, D), f32),
            pltpu.VMEM((2, _K), jnp.int32),
            pltpu.VMEM((2, _K), jnp.int32),
            pltpu.VMEM((_K, D), f32),
            pltpu.VMEM((_K, D), f32),
            pltpu.SemaphoreType.DMA,
            pltpu.SemaphoreType.DMA,
        ],
    )


@functools.lru_cache(maxsize=None)
def _make_sc_degrees(N, E):
    EPT = E // _NW
    NGRP = EPT // 16
    assert NGRP * 16 == EPT
    f32 = jnp.float32
    mesh = plsc.VectorSubcoreMesh(core_axis_name="c", subcore_axis_name="s")
    out_type = [jax.ShapeDtypeStruct((_NW * N,), f32)] * 3

    def body(dst_f, dst_b, dst_rb, zeros_deg,
             deg_f, deg_b, deg_rb,
             deg, dbulk):
        c = lax.axis_index("c")
        s = lax.axis_index("s")
        wid = c * _NS + s
        base = wid * EPT
        ones16 = jnp.ones((16,), f32)
        NGB = 2016
        assert EPT % NGB == 0 and NGB % 16 == 0
        for dsts, deg_out in ((dst_f, deg_f), (dst_b, deg_b),
                              (dst_rb, deg_rb)):
            pltpu.sync_copy(zeros_deg, deg)
            for m in range(EPT // NGB):
                pltpu.sync_copy(
                    dsts.at[pl.ds(pl.multiple_of(base + m * NGB, 8), NGB)],
                    dbulk)

                def grp(g, carry):
                    iv = dbulk[pl.ds(g * 16, 16)]
                    plsc.addupdate_scatter(deg, [iv], ones16)
                    return carry

                lax.fori_loop(0, NGB // 16, grp, 0)
            pltpu.sync_copy(deg, deg_out.at[pl.ds(wid * N, N)])

    return pl.kernel(
        body,
        out_type=out_type,
        mesh=mesh,
        compiler_params=pltpu.CompilerParams(needs_layout_passes=False),
        scratch_types=[
            pltpu.VMEM((N,), f32),
            pltpu.VMEM((2016,), jnp.int32),
        ],
    )


@functools.lru_cache(maxsize=None)
def _make_epilogue(N, Np, D, nrel):
    f32 = jnp.float32
    R = 128

    def body(*args):
        o_ref = args[-1]
        ones = jnp.ones((_NW, 1), f32)
        for r in range(nrel):
            a_ref, d_ref, w_ref = args[3 * r], args[3 * r + 1], args[3 * r + 2]
            p = a_ref[0] + a_ref[1]
            d = d_ref[:, 0, 0, :]
            dcol = lax.dot_general(d, ones, (((0,), (0,)), ((), ())),
                                   preferred_element_type=f32)
            dcol = jnp.maximum(dcol, 1.0)
            o_ref[:, r, :] = jnp.dot(p / dcol, w_ref[...],
                                     preferred_element_type=f32)

    in_specs = []
    for _ in range(nrel):
        in_specs += [
            pl.BlockSpec((_NC, R, D), lambda i: (0, i, 0)),
            pl.BlockSpec((_NW, 1, 1, 128), lambda i: (0, i, 0, 0)),
            pl.BlockSpec((D, D), lambda i: (0, 0)),
        ]
    grid = (pl.cdiv(N, R),)
    return pl.pallas_call(
        body,
        grid=grid,
        in_specs=in_specs,
        out_specs=pl.BlockSpec((R, nrel, D), lambda i: (i, 0, 0)),
        out_shape=jax.ShapeDtypeStruct((N, nrel, D), f32),
    )


def kernel(x_user, x_item, edge_index_follows, edge_index_buys,
           edge_index_rev_buys, W_follows, W_buys, W_rev_buys):
    N, D = x_user.shape
    E = edge_index_follows.shape[1]
    i32 = jnp.int32
    f32 = jnp.float32
    Np = _round_up(N, _K * _NS)
    Ep = _round_up(E, 2 * _K * _NW)
    npad = Ep - E
    pad_src = jnp.zeros((npad,), i32)
    pad_dst = N + (jnp.arange(npad, dtype=i32) % (Np - N))

    def prep(ei):
        return (jnp.concatenate([ei[0].astype(i32), pad_src]),
                jnp.concatenate([ei[1].astype(i32), pad_dst]))

    src_f, dst_f = prep(edge_index_follows)
    src_b, dst_b = prep(edge_index_buys)
    src_rb, dst_rb = prep(edge_index_rev_buys)
    zeros_feat = jnp.zeros((Np, D), f32)
    zeros_deg = jnp.zeros((Np,), f32)

    agg_f, agg_b, agg_rb = _make_sc_features(Np, D, Ep)(
        src_f, dst_f, src_b, dst_b, src_rb, dst_rb,
        x_user.astype(f32), x_item.astype(f32), zeros_feat)
    deg_f, deg_b, deg_rb = _make_sc_degrees(Np, Ep)(
        dst_f, dst_b, dst_rb, zeros_deg)
    deg_f, deg_b, deg_rb = (d.reshape(_NW, Np // 128, 1, 128)
                            for d in (deg_f, deg_b, deg_rb))

    out_user = _make_epilogue(N, Np, D, 2)(
        agg_f, deg_f, W_follows, agg_rb, deg_rb, W_rev_buys)
    out_item = _make_epilogue(N, Np, D, 1)(agg_b, deg_b, W_buys)
    return out_user, out_item


# speedup vs baseline: 1.8626x; 1.0881x over previous
import functools

import jax
import jax.numpy as jnp
from jax import lax
from jax.experimental import pallas as pl
from jax.experimental.pallas import tpu as pltpu
from jax.experimental.pallas import tpu_sc as plsc

_K = 80
_NC = 2
_NS = 16
_NW = _NC * _NS


def _round_up(x, m):
    return (x + m - 1) // m * m


@functools.lru_cache(maxsize=None)
def _make_sc_features(N, D, E):
    EPT = E // _NW
    CH = EPT // _K
    assert CH * _K == EPT and EPT * _NW == E
    RPT = N // _NS
    assert RPT % _K == 0
    NB = RPT // _K
    f32 = jnp.float32
    mesh = plsc.VectorSubcoreMesh(core_axis_name="c", subcore_axis_name="s")
    out_type = [jax.ShapeDtypeStruct((_NC, N, D), f32)] * 3

    assert CH % 2 == 0

    def body(src_f, dst_f, src_b, dst_b, src_rb, dst_rb,
             x_user, x_item, zeros_feat,
             agg_f, agg_b, agg_rb,
             acc, idx0, idx1, rows0, rows1, sem0, sem1, sem_i0, sem_i1):
        c = lax.axis_index("c")
        s = lax.axis_index("s")
        wid = c * _NS + s
        r0 = s * RPT
        dummy = zeros_feat.at[pl.ds(0, _K)]

        for srcs, dsts, table, agg_out in (
                (src_f, dst_f, x_user, agg_f),
                (src_b, dst_b, x_user, agg_b),
                (src_rb, dst_rb, x_item, agg_rb)):
            pltpu.sync_copy(zeros_feat.at[pl.ds(0, _K)], rows0)
            for z in range(NB):
                pltpu.sync_copy(rows0, acc.at[pl.ds(r0 + z * _K, _K)])
            plsc.subcore_barrier()
            base = wid * EPT

            idummy = srcs.at[pl.ds(0, _K)]

            def idx_start(idx, sem, q):
                off = pl.multiple_of(base + q * _K, 8)
                pltpu.async_copy(srcs.at[pl.ds(off, _K)], idx.at[0], sem)
                pltpu.async_copy(dsts.at[pl.ds(off, _K)], idx.at[1], sem)

            def idx_wait(idx, sem):
                pltpu.make_async_copy(idummy, idx.at[0], sem).wait()
                pltpu.make_async_copy(idummy, idx.at[1], sem).wait()

            idx_start(idx0, sem_i0, 0)
            idx_start(idx1, sem_i1, 1)
            idx_wait(idx0, sem_i0)
            pltpu.async_copy(table.at[idx0.at[0]], rows0, sem0)

            def inner(j2, carry):
                q0 = j2 * 2
                idx_wait(idx1, sem_i1)
                pltpu.async_copy(table.at[idx1.at[0]], rows1, sem1)
                pltpu.make_async_copy(dummy, rows0, sem0).wait()
                pltpu.sync_copy(rows0, acc.at[idx0.at[1]], add=True)

                @pl.when(q0 + 2 < CH)
                def _():
                    idx_start(idx0, sem_i0, q0 + 2)

                @pl.when(q0 + 2 < CH)
                def _():
                    idx_wait(idx0, sem_i0)
                    pltpu.async_copy(table.at[idx0.at[0]], rows0, sem0)

                pltpu.make_async_copy(dummy, rows1, sem1).wait()
                pltpu.sync_copy(rows1, acc.at[idx1.at[1]], add=True)

                @pl.when(q0 + 3 < CH)
                def _():
                    idx_start(idx1, sem_i1, q0 + 3)
                return carry

            lax.fori_loop(0, CH // 2, inner, 0)
            plsc.subcore_barrier()
            for z in range(NB):
                pltpu.sync_copy(acc.at[pl.ds(r0 + z * _K, _K)], rows0)
                pltpu.sync_copy(rows0, agg_out.at[c, pl.ds(r0 + z * _K, _K)])

    return pl.kernel(
        body,
        out_type=out_type,
        mesh=mesh,
        compiler_params=pltpu.CompilerParams(needs_layout_passes=False),
        scratch_types=[
            pltpu.VMEM_SHARED((N, D), f32),
            pltpu.VMEM((2, _K), jnp.int32),
            pltpu.VMEM((2, _K), jnp.int32),
            pltpu.VMEM((_K, D), f32),
            pltpu.VMEM((_K, D), f32),
            pltpu.SemaphoreType.DMA,
            pltpu.SemaphoreType.DMA,
            pltpu.SemaphoreType.DMA,
            pltpu.SemaphoreType.DMA,
        ],
    )


@functools.lru_cache(maxsize=None)
def _make_sc_degrees(N, E):
    EPT = E // _NW
    NGRP = EPT // 16
    assert NGRP * 16 == EPT
    f32 = jnp.float32
    mesh = plsc.VectorSubcoreMesh(core_axis_name="c", subcore_axis_name="s")
    out_type = [jax.ShapeDtypeStruct((_NW * N,), f32)] * 3

    def body(dst_f, dst_b, dst_rb, zeros_deg,
             deg_f, deg_b, deg_rb,
             deg, dbulk):
        c = lax.axis_index("c")
        s = lax.axis_index("s")
        wid = c * _NS + s
        base = wid * EPT
        ones16 = jnp.ones((16,), f32)
        NGB = 2016
        assert EPT % NGB == 0 and NGB % 16 == 0
        for dsts, deg_out in ((dst_f, deg_f), (dst_b, deg_b),
                              (dst_rb, deg_rb)):
            pltpu.sync_copy(zeros_deg, deg)
            for m in range(EPT // NGB):
                pltpu.sync_copy(
                    dsts.at[pl.ds(pl.multiple_of(base + m * NGB, 8), NGB)],
                    dbulk)

                def grp(g, carry):
                    iv = dbulk[pl.ds(g * 16, 16)]
                    plsc.addupdate_scatter(deg, [iv], ones16)
                    return carry

                lax.fori_loop(0, NGB // 16, grp, 0)
            pltpu.sync_copy(deg, deg_out.at[pl.ds(wid * N, N)])

    return pl.kernel(
        body,
        out_type=out_type,
        mesh=mesh,
        compiler_params=pltpu.CompilerParams(needs_layout_passes=False),
        scratch_types=[
            pltpu.VMEM((N,), f32),
            pltpu.VMEM((2016,), jnp.int32),
        ],
    )


@functools.lru_cache(maxsize=None)
def _make_epilogue(N, Np, D, nrel):
    f32 = jnp.float32
    R = 128

    def body(*args):
        o_ref = args[-1]
        ones = jnp.ones((_NW, 1), f32)
        for r in range(nrel):
            a_ref, d_ref, w_ref = args[3 * r], args[3 * r + 1], args[3 * r + 2]
            p = a_ref[0] + a_ref[1]
            d = d_ref[:, 0, 0, :]
            dcol = lax.dot_general(d, ones, (((0,), (0,)), ((), ())),
                                   preferred_element_type=f32)
            dcol = jnp.maximum(dcol, 1.0)
            o_ref[:, r, :] = jnp.dot(p / dcol, w_ref[...],
                                     preferred_element_type=f32)

    in_specs = []
    for _ in range(nrel):
        in_specs += [
            pl.BlockSpec((_NC, R, D), lambda i: (0, i, 0)),
            pl.BlockSpec((_NW, 1, 1, 128), lambda i: (0, i, 0, 0)),
            pl.BlockSpec((D, D), lambda i: (0, 0)),
        ]
    grid = (pl.cdiv(N, R),)
    return pl.pallas_call(
        body,
        grid=grid,
        in_specs=in_specs,
        out_specs=pl.BlockSpec((R, nrel, D), lambda i: (i, 0, 0)),
        out_shape=jax.ShapeDtypeStruct((N, nrel, D), f32),
    )


def kernel(x_user, x_item, edge_index_follows, edge_index_buys,
           edge_index_rev_buys, W_follows, W_buys, W_rev_buys):
    N, D = x_user.shape
    E = edge_index_follows.shape[1]
    i32 = jnp.int32
    f32 = jnp.float32
    Np = _round_up(N, _K * _NS)
    Ep = _round_up(E, 2 * _K * _NW)
    npad = Ep - E
    pad_src = jnp.zeros((npad,), i32)
    pad_dst = N + (jnp.arange(npad, dtype=i32) % (Np - N))

    def prep(ei):
        return (jnp.concatenate([ei[0].astype(i32), pad_src]),
                jnp.concatenate([ei[1].astype(i32), pad_dst]))

    src_f, dst_f = prep(edge_index_follows)
    src_b, dst_b = prep(edge_index_buys)
    src_rb, dst_rb = prep(edge_index_rev_buys)
    zeros_feat = jnp.zeros((Np, D), f32)
    zeros_deg = jnp.zeros((Np,), f32)

    agg_f, agg_b, agg_rb = _make_sc_features(Np, D, Ep)(
        src_f, dst_f, src_b, dst_b, src_rb, dst_rb,
        x_user.astype(f32), x_item.astype(f32), zeros_feat)
    deg_f, deg_b, deg_rb = _make_sc_degrees(Np, Ep)(
        dst_f, dst_b, dst_rb, zeros_deg)
    deg_f, deg_b, deg_rb = (d.reshape(_NW, Np // 128, 1, 128)
                            for d in (deg_f, deg_b, deg_rb))

    out_user = _make_epilogue(N, Np, D, 2)(
        agg_f, deg_f, W_follows, agg_rb, deg_rb, W_rev_buys)
    out_item = _make_epilogue(N, Np, D, 1)(agg_b, deg_b, W_buys)
    return out_user, out_item


# speedup vs baseline: 1.9582x; 1.0513x over previous
import functools

import jax
import jax.numpy as jnp
from jax import lax
from jax.experimental import pallas as pl
from jax.experimental.pallas import tpu as pltpu
from jax.experimental.pallas import tpu_sc as plsc

_K = 80
_NC = 2
_NS = 16
_NW = _NC * _NS


def _round_up(x, m):
    return (x + m - 1) // m * m


@functools.lru_cache(maxsize=None)
def _make_sc_features(N, D, E):
    EPT = E // _NW
    CH = EPT // _K
    assert CH * _K == EPT and EPT * _NW == E
    RPT = N // _NS
    assert RPT % _K == 0
    NB = RPT // _K
    f32 = jnp.float32
    mesh = plsc.VectorSubcoreMesh(core_axis_name="c", subcore_axis_name="s")
    out_type = [jax.ShapeDtypeStruct((_NC, N, D), f32)] * 3

    assert CH % 6 == 0

    def body(src_f, dst_f, src_b, dst_b, src_rb, dst_rb,
             x_user, x_item, zeros_feat,
             agg_f, agg_b, agg_rb,
             acc, idx0, idx1, idx2, idx3, idx4, idx5, rows0, rows1,
             sg0, sg1, ss0, ss1, si0, si1, si2, si3, si4, si5):
        c = lax.axis_index("c")
        s = lax.axis_index("s")
        wid = c * _NS + s
        r0 = s * RPT
        dummy = zeros_feat.at[pl.ds(0, _K)]
        idxs = (idx0, idx1, idx2, idx3, idx4, idx5)
        sis = (si0, si1, si2, si3, si4, si5)
        rws = (rows0, rows1)
        sgs = (sg0, sg1)
        sss = (ss0, ss1)

        for srcs, dsts, table, agg_out in (
                (src_f, dst_f, x_user, agg_f),
                (src_b, dst_b, x_user, agg_b),
                (src_rb, dst_rb, x_item, agg_rb)):
            pltpu.sync_copy(zeros_feat.at[pl.ds(0, _K)], rows0)
            for z in range(NB):
                pltpu.sync_copy(rows0, acc.at[pl.ds(r0 + z * _K, _K)])
            plsc.subcore_barrier()
            base = wid * EPT

            idummy = srcs.at[pl.ds(0, _K)]

            def idx_start(idx, sem, q):
                off = pl.multiple_of(base + q * _K, 8)
                pltpu.async_copy(srcs.at[pl.ds(off, _K)], idx.at[0], sem)
                pltpu.async_copy(dsts.at[pl.ds(off, _K)], idx.at[1], sem)

            def idx_wait(idx, sem):
                pltpu.make_async_copy(idummy, idx.at[0], sem).wait()
                pltpu.make_async_copy(idummy, idx.at[1], sem).wait()

            idx_start(idx0, si0, 0)
            idx_start(idx1, si1, 1)
            idx_wait(idx0, si0)
            pltpu.async_copy(table.at[idx0.at[0]], rows0, sg0)

            def inner(j6, carry):
                qb = j6 * 6
                for k in range(6):
                    q = qb + k
                    b = k % 2
                    @pl.when(q > 0)
                    def _():
                        pltpu.make_async_copy(dummy, rws[1 - b],
                                              sss[1 - b]).wait()

                    @pl.when(q + 1 < CH)
                    def _():
                        idx_wait(idxs[(k + 1) % 6], sis[(k + 1) % 6])
                        pltpu.async_copy(table.at[idxs[(k + 1) % 6].at[0]],
                                         rws[1 - b], sgs[1 - b])

                    pltpu.make_async_copy(dummy, rws[b], sgs[b]).wait()
                    pltpu.async_copy(rws[b], acc.at[idxs[k].at[1]],
                                     sss[b], add=True)

                    @pl.when(q + 2 < CH)
                    def _():
                        idx_start(idxs[(k + 2) % 6], sis[(k + 2) % 6], q + 2)
                return carry

            lax.fori_loop(0, CH // 6, inner, 0)
            pltpu.make_async_copy(dummy, rws[(CH - 1) % 2],
                                  sss[(CH - 1) % 2]).wait()
            plsc.subcore_barrier()
            for z in range(NB):
                pltpu.sync_copy(acc.at[pl.ds(r0 + z * _K, _K)], rows0)
                pltpu.sync_copy(rows0, agg_out.at[c, pl.ds(r0 + z * _K, _K)])

    return pl.kernel(
        body,
        out_type=out_type,
        mesh=mesh,
        compiler_params=pltpu.CompilerParams(needs_layout_passes=False),
        scratch_types=(
            [pltpu.VMEM_SHARED((N, D), f32)]
            + [pltpu.VMEM((2, _K), jnp.int32)] * 6
            + [pltpu.VMEM((_K, D), f32)] * 2
            + [pltpu.SemaphoreType.DMA] * 10
        ),
    )


@functools.lru_cache(maxsize=None)
def _make_sc_degrees(N, E):
    EPT = E // _NW
    NGRP = EPT // 16
    assert NGRP * 16 == EPT
    f32 = jnp.float32
    mesh = plsc.VectorSubcoreMesh(core_axis_name="c", subcore_axis_name="s")
    out_type = [jax.ShapeDtypeStruct((_NW * N,), f32)] * 3

    def body(dst_f, dst_b, dst_rb, zeros_deg,
             deg_f, deg_b, deg_rb,
             deg, dbulk):
        c = lax.axis_index("c")
        s = lax.axis_index("s")
        wid = c * _NS + s
        base = wid * EPT
        ones16 = jnp.ones((16,), f32)
        NGB = 2016
        assert EPT % NGB == 0 and NGB % 16 == 0
        for dsts, deg_out in ((dst_f, deg_f), (dst_b, deg_b),
                              (dst_rb, deg_rb)):
            pltpu.sync_copy(zeros_deg, deg)
            for m in range(EPT // NGB):
                pltpu.sync_copy(
                    dsts.at[pl.ds(pl.multiple_of(base + m * NGB, 8), NGB)],
                    dbulk)

                def grp(g, carry):
                    iv = dbulk[pl.ds(g * 16, 16)]
                    plsc.addupdate_scatter(deg, [iv], ones16)
                    return carry

                lax.fori_loop(0, NGB // 16, grp, 0)
            pltpu.sync_copy(deg, deg_out.at[pl.ds(wid * N, N)])

    return pl.kernel(
        body,
        out_type=out_type,
        mesh=mesh,
        compiler_params=pltpu.CompilerParams(needs_layout_passes=False),
        scratch_types=[
            pltpu.VMEM((N,), f32),
            pltpu.VMEM((2016,), jnp.int32),
        ],
    )


@functools.lru_cache(maxsize=None)
def _make_epilogue(N, Np, D, nrel):
    f32 = jnp.float32
    R = 128

    def body(*args):
        o_ref = args[-1]
        ones = jnp.ones((_NW, 1), f32)
        for r in range(nrel):
            a_ref, d_ref, w_ref = args[3 * r], args[3 * r + 1], args[3 * r + 2]
            p = a_ref[0] + a_ref[1]
            d = d_ref[:, 0, 0, :]
            dcol = lax.dot_general(d, ones, (((0,), (0,)), ((), ())),
                                   preferred_element_type=f32)
            dcol = jnp.maximum(dcol, 1.0)
            o_ref[:, r, :] = jnp.dot(p / dcol, w_ref[...],
                                     preferred_element_type=f32)

    in_specs = []
    for _ in range(nrel):
        in_specs += [
            pl.BlockSpec((_NC, R, D), lambda i: (0, i, 0)),
            pl.BlockSpec((_NW, 1, 1, 128), lambda i: (0, i, 0, 0)),
            pl.BlockSpec((D, D), lambda i: (0, 0)),
        ]
    grid = (pl.cdiv(N, R),)
    return pl.pallas_call(
        body,
        grid=grid,
        in_specs=in_specs,
        out_specs=pl.BlockSpec((R, nrel, D), lambda i: (i, 0, 0)),
        out_shape=jax.ShapeDtypeStruct((N, nrel, D), f32),
    )


def kernel(x_user, x_item, edge_index_follows, edge_index_buys,
           edge_index_rev_buys, W_follows, W_buys, W_rev_buys):
    N, D = x_user.shape
    E = edge_index_follows.shape[1]
    i32 = jnp.int32
    f32 = jnp.float32
    Np = _round_up(N, _K * _NS)
    Ep = _round_up(E, 6 * _K * _NW)
    npad = Ep - E
    pad_src = jnp.zeros((npad,), i32)
    pad_dst = N + (jnp.arange(npad, dtype=i32) % (Np - N))

    def prep(ei):
        return (jnp.concatenate([ei[0].astype(i32), pad_src]),
                jnp.concatenate([ei[1].astype(i32), pad_dst]))

    src_f, dst_f = prep(edge_index_follows)
    src_b, dst_b = prep(edge_index_buys)
    src_rb, dst_rb = prep(edge_index_rev_buys)
    zeros_feat = jnp.zeros((Np, D), f32)
    zeros_deg = jnp.zeros((Np,), f32)

    agg_f, agg_b, agg_rb = _make_sc_features(Np, D, Ep)(
        src_f, dst_f, src_b, dst_b, src_rb, dst_rb,
        x_user.astype(f32), x_item.astype(f32), zeros_feat)
    deg_f, deg_b, deg_rb = _make_sc_degrees(Np, Ep)(
        dst_f, dst_b, dst_rb, zeros_deg)
    deg_f, deg_b, deg_rb = (d.reshape(_NW, Np // 128, 1, 128)
                            for d in (deg_f, deg_b, deg_rb))

    out_user = _make_epilogue(N, Np, D, 2)(
        agg_f, deg_f, W_follows, agg_rb, deg_rb, W_rev_buys)
    out_item = _make_epilogue(N, Np, D, 1)(agg_b, deg_b, W_buys)
    return out_user, out_item
